# Initial kernel scaffold; baseline (speedup 1.0000x reference)
import jax
import jax.numpy as jnp
from jax.experimental import pallas as pl


def kernel(feat, W_in, b_in, Wg1, bg1, Wg2, bg2, Wg3, bg3, Ws1, bs1, Ws2, bs2, We1, be1, We2, be2, edge_index, degrees, num_nodes):
    raise NotImplementedError("write your pallas kernel here")


# speedup vs baseline: 39.7078x; 39.7078x over previous
import functools

import jax
import jax.numpy as jnp
from jax import lax
from jax.experimental import pallas as pl
from jax.experimental.pallas import tpu as pltpu
from jax.experimental.pallas import tpu_sc as plsc

NC = 2
NS = 16
NW = NC * NS
CHUNK = 128
SLOTS = 4
PRE = 2
BLK = 1280


def _mesh():
    return plsc.VectorSubcoreMesh(
        core_axis_name="c", subcore_axis_name="s", num_cores=NC, num_subcores=NS
    )


def _deg_pass(dst3, ones_c, zeros_n, npad, nchunk):
    rpt = npad // NS
    K = 8

    @functools.partial(
        pl.kernel,
        out_type=jax.ShapeDtypeStruct((NC, npad), jnp.float32),
        mesh=_mesh(),
        compiler_params=pltpu.CompilerParams(use_tc_tiling_on_sc=False),
        scratch_types=[
            pltpu.VMEM((nchunk, CHUNK), jnp.int32),
            pltpu.VMEM((CHUNK,), jnp.float32),
            pltpu.VMEM_SHARED((npad,), jnp.float32),
            pltpu.SemaphoreType.DMA,
        ],
    )
    def deg(dst_hbm, ones_hbm, zeros_hbm, out_hbm, dst_v, ones_v, acc, sem):
        c = lax.axis_index("c")
        s = lax.axis_index("s")
        w = c * NS + s
        pltpu.sync_copy(zeros_hbm.at[pl.ds(s * rpt, rpt)], acc.at[pl.ds(s * rpt, rpt)])
        pltpu.sync_copy(ones_hbm, ones_v)
        pltpu.sync_copy(dst_hbm.at[w], dst_v)
        plsc.subcore_barrier()
        for g in range(0, nchunk, K):
            cps = [
                pltpu.async_copy(ones_v, acc.at[dst_v.at[j]], sem, add=True)
                for j in range(g, min(g + K, nchunk))
            ]
            for cp in cps:
                cp.wait()
        plsc.subcore_barrier()
        pltpu.sync_copy(acc.at[pl.ds(s * rpt, rpt)], out_hbm.at[c, pl.ds(s * rpt, rpt)])

    return deg(dst3, ones_c, zeros_n)


def _edge_pass(t, src3, dst3, zeros_nc, npad, nchunk, C):
    rpt = npad // NS

    @functools.partial(
        pl.kernel,
        out_type=jax.ShapeDtypeStruct((NC, npad, C), jnp.float32),
        mesh=_mesh(),
        compiler_params=pltpu.CompilerParams(use_tc_tiling_on_sc=False),
        scratch_types=[
            pltpu.VMEM((nchunk, CHUNK), jnp.int32),
            pltpu.VMEM((nchunk, CHUNK), jnp.int32),
            pltpu.VMEM((SLOTS, CHUNK, C), jnp.float32),
            pltpu.VMEM_SHARED((npad, C), jnp.float32),
            pltpu.SemaphoreType.DMA,
            pltpu.SemaphoreType.DMA,
            pltpu.SemaphoreType.DMA,
            pltpu.SemaphoreType.DMA,
            pltpu.SemaphoreType.DMA,
            pltpu.SemaphoreType.DMA,
            pltpu.SemaphoreType.DMA,
            pltpu.SemaphoreType.DMA,
        ],
    )
    def layer(t_hbm, src_hbm, dst_hbm, zeros_hbm, out_hbm,
              src_v, dst_v, rows, acc, g0, g1, g2, g3, s0, s1, s2, s3):
        gs = [g0, g1, g2, g3]
        ss = [s0, s1, s2, s3]
        c = lax.axis_index("c")
        s = lax.axis_index("s")
        w = c * NS + s
        pltpu.sync_copy(zeros_hbm.at[pl.ds(s * rpt, rpt)], acc.at[pl.ds(s * rpt, rpt)])
        pltpu.sync_copy(src_hbm.at[w], src_v)
        pltpu.sync_copy(dst_hbm.at[w], dst_v)
        plsc.subcore_barrier()

        gcp = {}
        scp = {}

        def fire_gather(j):
            sl = j % SLOTS
            gcp[j] = pltpu.async_copy(t_hbm.at[src_v.at[j]], rows.at[sl], gs[sl])

        for j in range(min(PRE, nchunk)):
            fire_gather(j)
        for j in range(nchunk):
            nj = j + PRE
            if nj < nchunk:
                pj = nj - SLOTS
                if pj >= 0:
                    scp[pj].wait()
                fire_gather(nj)
            gcp[j].wait()
            sl = j % SLOTS
            scp[j] = pltpu.async_copy(rows.at[sl], acc.at[dst_v.at[j]], ss[sl], add=True)
        for j in range(max(0, nchunk - SLOTS), nchunk):
            scp[j].wait()
        plsc.subcore_barrier()
        pltpu.sync_copy(acc.at[pl.ds(s * rpt, rpt)], out_hbm.at[c, pl.ds(s * rpt, rpt)])

    return layer(t, src3, dst3, zeros_nc)


def _relu6(v):
    return jnp.clip(v, 0.0, 6.0)


def _dot(a, b):
    return jnp.dot(a, b, preferred_element_type=jnp.float32)


def _rows_spec(c):
    return pl.BlockSpec((BLK, c), lambda i: (i, 0))


def _bcast_spec(shape):
    nd = len(shape)
    return pl.BlockSpec(shape, lambda i: (0,) * nd)


def _tc_fc_in(x, W, b):
    npad, k = x.shape

    def body(x_ref, w_ref, b_ref, o_ref):
        o_ref[...] = _relu6(_dot(x_ref[...], w_ref[...]) + b_ref[...])

    return pl.pallas_call(
        body,
        grid=(npad // BLK,),
        in_specs=[_rows_spec(k), _bcast_spec(W.shape), _bcast_spec(b.shape)],
        out_specs=_rows_spec(W.shape[1]),
        out_shape=jax.ShapeDtypeStruct((npad, W.shape[1]), jnp.float32),
    )(x, W, b)


def _tc_dinv_t1(degp, h1, Wg1):
    npad, C = h1.shape

    def body(d_ref, h_ref, w_ref, dinv_ref, t_ref):
        d = d_ref[0] + d_ref[1] + 1.0
        dinv = 1.0 / jnp.sqrt(d)
        dinv_ref[...] = dinv
        t_ref[...] = _dot(h_ref[...], w_ref[...]) * dinv

    return pl.pallas_call(
        body,
        grid=(npad // BLK,),
        in_specs=[
            pl.BlockSpec((2, BLK, 1), lambda i: (0, i, 0)),
            _rows_spec(C),
            _bcast_spec(Wg1.shape),
        ],
        out_specs=(_rows_spec(1), _rows_spec(Wg1.shape[1])),
        out_shape=(
            jax.ShapeDtypeStruct((npad, 1), jnp.float32),
            jax.ShapeDtypeStruct((npad, Wg1.shape[1]), jnp.float32),
        ),
    )(degp, h1, Wg1)


def _tc_mid(P, t, dinv, b, Wn):
    npad, C = t.shape

    def body(p_ref, t_ref, di_ref, b_ref, w_ref, o_ref):
        di = di_ref[...]
        u = di * (p_ref[0] + p_ref[1] + t_ref[...])
        h = _relu6(u + b_ref[...])
        o_ref[...] = _dot(h, w_ref[...]) * di

    return pl.pallas_call(
        body,
        grid=(npad // BLK,),
        in_specs=[
            pl.BlockSpec((2, BLK, C), lambda i: (0, i, 0)),
            _rows_spec(C),
            _rows_spec(1),
            _bcast_spec(b.shape),
            _bcast_spec(Wn.shape),
        ],
        out_specs=_rows_spec(Wn.shape[1]),
        out_shape=jax.ShapeDtypeStruct((npad, Wn.shape[1]), jnp.float32),
    )(P, t, dinv, b, Wn)


def _tc_head_dense(P, t3, dinv, bg3, Ws1, bs1, Ws2, bs2, We1_top):
    npad, C = t3.shape

    def body(p_ref, t_ref, di_ref, bg_ref, ws1_ref, bs1_ref, ws2_ref,
             bs2_ref, we_ref, h4_ref, s_ref, g_ref):
        di = di_ref[...]
        u = di * (p_ref[0] + p_ref[1] + t_ref[...])
        h4 = _relu6(u + bg_ref[...])
        h4_ref[...] = h4
        s1 = jnp.maximum(_dot(h4, ws1_ref[...]) + bs1_ref[...], 0.0)
        s_ref[...] = _dot(s1, ws2_ref[...]) + bs2_ref[...]
        g_ref[...] = _dot(h4, we_ref[...])

    ne = We1_top.shape[1]
    return pl.pallas_call(
        body,
        grid=(npad // BLK,),
        in_specs=[
            pl.BlockSpec((2, BLK, C), lambda i: (0, i, 0)),
            _rows_spec(C),
            _rows_spec(1),
            _bcast_spec(bg3.shape),
            _bcast_spec(Ws1.shape),
            _bcast_spec(bs1.shape),
            _bcast_spec(Ws2.shape),
            _bcast_spec(bs2.shape),
            _bcast_spec(We1_top.shape),
        ],
        out_specs=(_rows_spec(C), _rows_spec(1), _rows_spec(ne)),
        out_shape=(
            jax.ShapeDtypeStruct((npad, C), jnp.float32),
            jax.ShapeDtypeStruct((npad, 1), jnp.float32),
            jax.ShapeDtypeStruct((npad, ne), jnp.float32),
        ),
    )(P, t3, dinv, bg3, Ws1, bs1, Ws2, bs2, We1_top)


def _tc_head_final(h4, s_raw, g, We1_bot, be1, We2, be2, n_nodes, n_total):
    npad = h4.shape[0]

    def body(h4_ref, s_ref, g_ref, web_ref, be1_ref, we2_ref, be2_ref, o_ref):
        s_raw_v = s_ref[...]
        ridx = lax.broadcasted_iota(jnp.int32, (npad, 1), 0)
        valid = ridx < n_total
        neg = jnp.float32(-1e30)
        smax = jnp.max(jnp.where(valid, s_raw_v, neg))
        s_out = s_raw_v - smax
        allowed = valid & ((ridx < n_nodes) | (ridx == n_total - 1))
        sm = jnp.where(allowed, s_out, neg)
        amax = jnp.max(sm)
        a = jnp.min(jnp.where(allowed & (sm >= amax), ridx, jnp.int32(npad)))
        ha = jnp.sum(jnp.where(ridx == a, h4_ref[...], 0.0), axis=0,
                     keepdims=True)
        e1 = jnp.maximum(g_ref[...] + _dot(ha, web_ref[...]) + be1_ref[...], 0.0)
        e_raw = _dot(e1, we2_ref[...]) + be2_ref[...]
        emax = jnp.max(jnp.where(valid, e_raw, neg))
        e_out = e_raw - emax
        o_ref[...] = jnp.concatenate([s_out, e_out], axis=1)

    return pl.pallas_call(
        body,
        out_shape=jax.ShapeDtypeStruct((npad, 2), jnp.float32),
    )(h4, s_raw, g, We1_bot, be1, We2, be2)


def kernel(feat, W_in, b_in, Wg1, bg1, Wg2, bg2, Wg3, bg3, Ws1, bs1, Ws2, bs2,
           We1, be1, We2, be2, edge_index, degrees, num_nodes):
    n_nodes, nfeat = feat.shape
    n_total = n_nodes + nfeat
    npad = ((n_total + CHUNK - 1) // CHUNK) * CHUNK
    E = edge_index.shape[1]
    nchunk = (E + NW * CHUNK - 1) // (NW * CHUNK)
    e_pad = nchunk * NW * CHUNK

    f32 = jnp.float32
    cand = jnp.eye(nfeat, dtype=f32)
    x = jnp.concatenate([feat, cand, jnp.zeros((npad - n_total, nfeat), f32)], axis=0)

    pad = e_pad - E
    pad_ar = jnp.arange(pad, dtype=jnp.int32)
    src = jnp.concatenate([edge_index[0], pad_ar % n_nodes])
    dst = jnp.concatenate([edge_index[1], n_total + pad_ar % (npad - n_total)])
    src3 = src.reshape(NW, nchunk, CHUNK)
    dst3 = dst.reshape(NW, nchunk, CHUNK)

    ones_c = jnp.ones((CHUNK,), f32)
    zeros_n = jnp.zeros((npad,), f32)

    degp = _deg_pass(dst3, ones_c, zeros_n, npad, nchunk)
    h1 = _tc_fc_in(x, W_in, b_in.reshape(1, -1))
    dinv, t1 = _tc_dinv_t1(degp.reshape(NC, npad, 1), h1, Wg1)

    c1 = t1.shape[1]
    P1 = _edge_pass(t1, src3, dst3, jnp.zeros((npad, c1), f32), npad, nchunk, c1)
    t2 = _tc_mid(P1, t1, dinv, bg1.reshape(1, -1), Wg2)

    c2 = t2.shape[1]
    P2 = _edge_pass(t2, src3, dst3, jnp.zeros((npad, c2), f32), npad, nchunk, c2)
    t3 = _tc_mid(P2, t2, dinv, bg2.reshape(1, -1), Wg3)

    c3 = t3.shape[1]
    P3 = _edge_pass(t3, src3, dst3, jnp.zeros((npad, c3), f32), npad, nchunk, c3)

    hid = Wg3.shape[1]
    h4, s_raw, g = _tc_head_dense(
        P3, t3, dinv, bg3.reshape(1, -1), Ws1, bs1.reshape(1, -1),
        Ws2, bs2.reshape(1, -1), We1[:hid],
    )
    out_full = _tc_head_final(
        h4, s_raw, g, We1[hid:], be1.reshape(1, -1), We2, be2.reshape(1, -1),
        n_nodes, n_total,
    )
    return out_full[:n_total]
